# baseline (speedup 1.0000x reference)
import jax
import jax.numpy as jnp
from jax import lax
from jax.experimental import pallas as pl
from jax.experimental.pallas import tpu as pltpu
from jax.experimental.pallas import tpu_sc as plsc

BATCH = 16384
NF = 32
HID = 128
NC, NS = 2, 16
NW = NC * NS
BPW = BATCH // NW
CHUNK = 128
NCHUNK = BPW // CHUNK


def _gather_body(u_tab, m_tab, users, movies, ue_out, me_out,
                 uidx_v, midx_v, urows_v, mrows_v, sem):
    wid = lax.axis_index("s") * NC + lax.axis_index("c")
    base = wid * BPW
    pltpu.sync_copy(users.at[wid], uidx_v)
    pltpu.sync_copy(movies.at[wid], midx_v)
    copies = []
    for j in range(NCHUNK):
        copies.append(pltpu.async_copy(
            u_tab.at[uidx_v.at[j]], urows_v.at[pl.ds(j * CHUNK, CHUNK)], sem))
        copies.append(pltpu.async_copy(
            m_tab.at[midx_v.at[j]], mrows_v.at[pl.ds(j * CHUNK, CHUNK)], sem))
    for c in copies:
        c.wait()
    pltpu.sync_copy(urows_v, ue_out.at[pl.ds(base, BPW)])
    pltpu.sync_copy(mrows_v, me_out.at[pl.ds(base, BPW)])


@jax.jit
def _sc_gather(u_table, m_table, users, movies):
    mesh = plsc.VectorSubcoreMesh(core_axis_name="c", subcore_axis_name="s",
                                  num_cores=NC, num_subcores=NS)
    out_type = (jax.ShapeDtypeStruct((BATCH, NF), jnp.float32),
                jax.ShapeDtypeStruct((BATCH, NF), jnp.float32))
    scratch = [
        pltpu.VMEM((NCHUNK, CHUNK), jnp.int32),
        pltpu.VMEM((NCHUNK, CHUNK), jnp.int32),
        pltpu.VMEM((BPW, NF), jnp.float32),
        pltpu.VMEM((BPW, NF), jnp.float32),
        pltpu.SemaphoreType.DMA,
    ]
    users3 = users.astype(jnp.int32).reshape(NW, NCHUNK, CHUNK)
    movies3 = movies.astype(jnp.int32).reshape(NW, NCHUNK, CHUNK)
    return pl.kernel(_gather_body, out_type=out_type, mesh=mesh,
                     scratch_types=scratch,
                     compiler_params=pltpu.CompilerParams(
                         use_tc_tiling_on_sc=False))(
                             u_table, m_table, users3, movies3)


def _mlp_body(ue_ref, me_ref, w1u_ref, w1m_ref, b1_ref, w2_ref, b2_ref,
              wf_ref, bf_ref, out_ref):
    x = ue_ref[...] @ w1u_ref[...] + me_ref[...] @ w1m_ref[...] + b1_ref[...]
    x = jnp.maximum(x, 0.0)
    x = jnp.maximum(x @ w2_ref[...] + b2_ref[...], 0.0)
    z = jnp.sum(x * wf_ref[...], axis=1, keepdims=True) + bf_ref[0, 0]
    out_ref[...] = jax.nn.sigmoid(z) * 4.0 + 1.0


def _mlp(ue, me, W1, b1, W2, b2, Wf, bf):
    BM = 2048
    grid = (BATCH // BM,)
    w1u = W1[:NF]
    w1m = W1[NF:]
    b1r = b1.reshape(1, HID)
    b2r = b2.reshape(1, HID)
    wfr = Wf.reshape(1, HID)
    bfr = bf.reshape(1, 1)
    full = lambda shape: pl.BlockSpec(shape, lambda i: (0, 0))
    return pl.pallas_call(
        _mlp_body,
        grid=grid,
        in_specs=[
            pl.BlockSpec((BM, NF), lambda i: (i, 0)),
            pl.BlockSpec((BM, NF), lambda i: (i, 0)),
            full((NF, HID)),
            full((NF, HID)),
            full((1, HID)),
            full((HID, HID)),
            full((1, HID)),
            full((1, HID)),
            full((1, 1)),
        ],
        out_specs=pl.BlockSpec((BM, 1), lambda i: (i, 0)),
        out_shape=jax.ShapeDtypeStruct((BATCH, 1), jnp.float32),
        compiler_params=pltpu.CompilerParams(
            dimension_semantics=("arbitrary",)),
    )(ue, me, w1u, w1m, b1r, W2, b2r, wfr, bfr)


def kernel(users, movies, u_table, m_table, W1, b1, W2, b2, Wf, bf):
    ue, me = _sc_gather(u_table, m_table, users, movies)
    return _mlp(ue, me, W1, b1, W2, b2, Wf, bf)


# speedup vs baseline: 9.8843x; 9.8843x over previous
import jax
import jax.numpy as jnp
from jax.experimental import pallas as pl
from jax.experimental.pallas import tpu as pltpu

BATCH = 16384
NF = 32
HID = 128
BM = 2048


def _mlp_body(ue_ref, me_ref, w1u_ref, w1m_ref, b1_ref,
              w2_ref, b2_ref, wf_ref, bf_ref, out_ref):
    x = ue_ref[...] @ w1u_ref[...] + me_ref[...] @ w1m_ref[...] + b1_ref[...]
    x = jnp.maximum(x, 0.0)
    x = jnp.maximum(x @ w2_ref[...] + b2_ref[...], 0.0)
    z = jnp.sum(x * wf_ref[...], axis=1, keepdims=True) + bf_ref[0, 0]
    out_ref[...] = jax.nn.sigmoid(z) * 4.0 + 1.0


def _mlp(ue, me, W1, b1, W2, b2, Wf, bf):
    grid = (BATCH // BM,)
    w1u = W1[:NF]
    w1m = W1[NF:]
    b1r = b1.reshape(1, HID)
    b2r = b2.reshape(1, HID)
    wfr = Wf.reshape(1, HID)
    bfr = bf.reshape(1, 1)
    full = lambda shape: pl.BlockSpec(shape, lambda i: (0, 0))
    return pl.pallas_call(
        _mlp_body,
        grid=grid,
        in_specs=[
            pl.BlockSpec((BM, NF), lambda i: (i, 0)),
            pl.BlockSpec((BM, NF), lambda i: (i, 0)),
            full((NF, HID)),
            full((NF, HID)),
            full((1, HID)),
            full((HID, HID)),
            full((1, HID)),
            full((1, HID)),
            full((1, 1)),
        ],
        out_specs=pl.BlockSpec((BM, 1), lambda i: (i, 0)),
        out_shape=jax.ShapeDtypeStruct((BATCH, 1), jnp.float32),
        compiler_params=pltpu.CompilerParams(
            dimension_semantics=("arbitrary",)),
    )(ue, me, w1u, w1m, b1r, W2, b2r, wfr, bfr)


def kernel(users, movies, u_table, m_table, W1, b1, W2, b2, Wf, bf):
    ue = jnp.take(u_table, users, axis=0, mode="clip")
    me = jnp.take(m_table, movies, axis=0, mode="clip")
    return _mlp(ue, me, W1, b1, W2, b2, Wf, bf)
